# Initial kernel scaffold; baseline (speedup 1.0000x reference)
import jax
import jax.numpy as jnp
from jax.experimental import pallas as pl


def kernel(edge_index_1, edge_index_2, feature, W1, b1, W2, b2, W3, b3, W4, b4, W5, b5, W6, b6):
    raise NotImplementedError("write your pallas kernel here")


# speedup vs baseline: 12.8399x; 12.8399x over previous
import functools

import jax
import jax.numpy as jnp
from jax.experimental import pallas as pl
from jax.experimental.pallas import tpu as pltpu

N = 4096
D = 512


def _mm_body(a_ref, b_ref, scale_ref, bias_ref, out_ref, acc_ref, *,
             trans_lhs, scale_rows, bias, leaky, bm):
    k = pl.program_id(2)

    @pl.when(k == 0)
    def _():
        acc_ref[...] = jnp.zeros_like(acc_ref)

    if trans_lhs:
        acc_ref[...] += jax.lax.dot_general(
            a_ref[...], b_ref[...], (((0,), (0,)), ((), ())),
            preferred_element_type=jnp.float32)
    else:
        acc_ref[...] += jnp.dot(a_ref[...], b_ref[...],
                                preferred_element_type=jnp.float32)

    @pl.when(k == pl.num_programs(2) - 1)
    def _():
        acc = acc_ref[...]
        if bias:
            j = pl.program_id(1)
            bn = out_ref.shape[1]
            acc = acc + bias_ref[pl.ds(j * bn, bn)][None, :]
        if scale_rows:
            i = pl.program_id(0)
            acc = acc * scale_ref[pl.ds(i * bm, bm)][:, None]
        if leaky:
            acc = jnp.where(acc > 0, acc, 0.01 * acc)
        out_ref[...] = acc


def _mm(a, b, *, scale=None, bias=None, leaky=False, trans_lhs=False,
        bm=512, bn=512, bk=512):
    if trans_lhs:
        ka, m = a.shape
    else:
        m, ka = a.shape
    kb, n = b.shape
    assert ka == kb
    grid = (m // bm, n // bn, ka // bk)
    in_specs = [
        pl.BlockSpec((bk, bm) if trans_lhs else (bm, bk),
                     (lambda i, j, k: (k, i)) if trans_lhs
                     else (lambda i, j, k: (i, k))),
        pl.BlockSpec((bk, bn), lambda i, j, k: (k, j)),
        pl.BlockSpec((m,), lambda i, j, k: (0,)),
        pl.BlockSpec((n,), lambda i, j, k: (0,)),
    ]
    scale_arr = scale if scale is not None else jnp.zeros((m,), jnp.float32)
    bias_arr = bias if bias is not None else jnp.zeros((n,), jnp.float32)
    body = functools.partial(_mm_body, trans_lhs=trans_lhs,
                             scale_rows=scale is not None,
                             bias=bias is not None, leaky=leaky, bm=bm)
    return pl.pallas_call(
        body,
        grid=grid,
        in_specs=in_specs,
        out_specs=pl.BlockSpec((bm, bn), lambda i, j, k: (i, j)),
        out_shape=jax.ShapeDtypeStruct((m, n), jnp.float32),
        scratch_shapes=[pltpu.VMEM((bm, bn), jnp.float32)],
        compiler_params=pltpu.CompilerParams(
            dimension_semantics=("parallel", "parallel", "arbitrary")),
    )(a, b, scale_arr, bias_arr)


def _agg_body(a_ref, t_ref, td_ref, dis_ref, bias_ref, out_ref, acc_ref, *,
              leaky, bm):
    k = pl.program_id(2)

    @pl.when(k == 0)
    def _():
        acc_ref[...] = jnp.zeros_like(acc_ref)

    acc_ref[...] += jnp.dot(a_ref[...], t_ref[...],
                            preferred_element_type=jnp.float32)

    @pl.when(k == pl.num_programs(2) - 1)
    def _():
        i = pl.program_id(0)
        acc = acc_ref[...] + td_ref[...]
        acc = acc * dis_ref[pl.ds(i * bm, bm)][:, None]
        acc = acc + bias_ref[...][None, :]
        if leaky:
            acc = jnp.where(acc > 0, acc, 0.01 * acc)
        out_ref[...] = acc


def _agg(a0, ts, dis, bias, *, leaky, bm=512, bk=512):
    n, d = ts.shape
    grid = (n // bm, 1, n // bk)
    body = functools.partial(_agg_body, leaky=leaky, bm=bm)
    return pl.pallas_call(
        body,
        grid=grid,
        in_specs=[
            pl.BlockSpec((bm, bk), lambda i, j, k: (i, k)),
            pl.BlockSpec((bk, d), lambda i, j, k: (k, j)),
            pl.BlockSpec((bm, d), lambda i, j, k: (i, j)),
            pl.BlockSpec((n,), lambda i, j, k: (0,)),
            pl.BlockSpec((d,), lambda i, j, k: (0,)),
        ],
        out_specs=pl.BlockSpec((bm, d), lambda i, j, k: (i, j)),
        out_shape=jax.ShapeDtypeStruct((n, d), jnp.float32),
        scratch_shapes=[pltpu.VMEM((bm, d), jnp.float32)],
        compiler_params=pltpu.CompilerParams(
            dimension_semantics=("parallel", "parallel", "arbitrary")),
    )(a0, ts, ts, dis, bias)


def _rowscale_body(t_ref, dis_ref, out_ref, *, bm):
    i = pl.program_id(0)
    out_ref[...] = t_ref[...] * dis_ref[pl.ds(i * bm, bm)][:, None]


def _rowscale(t, dis, *, bm=512):
    n, d = t.shape
    return pl.pallas_call(
        functools.partial(_rowscale_body, bm=bm),
        grid=(n // bm,),
        in_specs=[pl.BlockSpec((bm, d), lambda i: (i, 0)),
                  pl.BlockSpec((n,), lambda i: (0,))],
        out_specs=pl.BlockSpec((bm, d), lambda i: (i, 0)),
        out_shape=jax.ShapeDtypeStruct((n, d), jnp.float32),
    )(t, dis)


def _dis_body(a_ref, out_ref):
    out_ref[...] = jax.lax.rsqrt(jnp.sum(a_ref[...], axis=1) + 1.0)


def _dis(a0, *, bm=512):
    n = a0.shape[0]
    return pl.pallas_call(
        _dis_body,
        grid=(n // bm,),
        in_specs=[pl.BlockSpec((bm, n), lambda i: (i, 0))],
        out_specs=pl.BlockSpec((bm,), lambda i: (i,)),
        out_shape=jax.ShapeDtypeStruct((n,), jnp.float32),
    )(a0)


def _conv_pass(t1, a0, params):
    (_, b1, W2, b2, W3, b3, W4, b4, W5, b5, W6, b6) = params
    dis = _dis(a0)
    f = _agg(a0, _rowscale(t1, dis), dis, b1, leaky=False)
    for (W, b) in ((W2, b2), (W3, b3), (W4, b4), (W5, b5)):
        ts = _mm(f, W, scale=dis)
        f = _agg(a0, ts, dis, b, leaky=True)
    g = _mm(a0, f)
    return _mm(g, W6, bias=b6, leaky=True)


def _build_a0(src, dst):
    return jnp.zeros((N, N), jnp.float32).at[dst, src].add(1.0)


def kernel(edge_index_1, edge_index_2, feature, W1, b1, W2, b2, W3, b3,
           W4, b4, W5, b5, W6, b6):
    params = (W1, b1, W2, b2, W3, b3, W4, b4, W5, b5, W6, b6)
    a0_1 = _build_a0(edge_index_1[0], edge_index_1[1])
    a0_2 = _build_a0(edge_index_2[0], edge_index_2[1])
    t1 = _mm(feature, W1)
    fa = _conv_pass(t1, a0_1, params)
    fb = _conv_pass(t1, a0_2, params)
    pred = _mm(fa, fb, trans_lhs=True)
    return (fa, fb, pred)


# speedup vs baseline: 14.3845x; 1.1203x over previous
import functools

import jax
import jax.numpy as jnp
from jax import lax
from jax.experimental import pallas as pl
from jax.experimental.pallas import tpu as pltpu
from jax.experimental.pallas import tpu_sc as plsc

N = 4096
D = 512
E = 131072


def _mm_body(a_ref, b_ref, scale_ref, bias_ref, out_ref, acc_ref, *,
             trans_lhs, scale_rows, bias, leaky, bm):
    k = pl.program_id(2)

    @pl.when(k == 0)
    def _():
        acc_ref[...] = jnp.zeros_like(acc_ref)

    if trans_lhs:
        acc_ref[...] += jax.lax.dot_general(
            a_ref[...], b_ref[...], (((0,), (0,)), ((), ())),
            preferred_element_type=jnp.float32)
    else:
        acc_ref[...] += jnp.dot(a_ref[...], b_ref[...],
                                preferred_element_type=jnp.float32)

    @pl.when(k == pl.num_programs(2) - 1)
    def _():
        acc = acc_ref[...]
        if bias:
            j = pl.program_id(1)
            bn = out_ref.shape[1]
            acc = acc + bias_ref[pl.ds(j * bn, bn)][None, :]
        if scale_rows:
            i = pl.program_id(0)
            acc = acc * scale_ref[pl.ds(i * bm, bm)][:, None]
        if leaky:
            acc = jnp.where(acc > 0, acc, 0.01 * acc)
        out_ref[...] = acc


def _mm(a, b, *, scale=None, bias=None, leaky=False, trans_lhs=False,
        bm=512, bn=512, bk=512):
    if trans_lhs:
        ka, m = a.shape
    else:
        m, ka = a.shape
    kb, n = b.shape
    assert ka == kb
    grid = (m // bm, n // bn, ka // bk)
    in_specs = [
        pl.BlockSpec((bk, bm) if trans_lhs else (bm, bk),
                     (lambda i, j, k: (k, i)) if trans_lhs
                     else (lambda i, j, k: (i, k))),
        pl.BlockSpec((bk, bn), lambda i, j, k: (k, j)),
        pl.BlockSpec((m,), lambda i, j, k: (0,)),
        pl.BlockSpec((n,), lambda i, j, k: (0,)),
    ]
    scale_arr = scale if scale is not None else jnp.zeros((m,), jnp.float32)
    bias_arr = bias if bias is not None else jnp.zeros((n,), jnp.float32)
    body = functools.partial(_mm_body, trans_lhs=trans_lhs,
                             scale_rows=scale is not None,
                             bias=bias is not None, leaky=leaky, bm=bm)
    return pl.pallas_call(
        body,
        grid=grid,
        in_specs=in_specs,
        out_specs=pl.BlockSpec((bm, bn), lambda i, j, k: (i, j)),
        out_shape=jax.ShapeDtypeStruct((m, n), jnp.float32),
        scratch_shapes=[pltpu.VMEM((bm, bn), jnp.float32)],
        compiler_params=pltpu.CompilerParams(
            dimension_semantics=("parallel", "parallel", "arbitrary")),
    )(a, b, scale_arr, bias_arr)


def _agg_body(a_ref, t_ref, td_ref, dis_ref, bias_ref, out_ref, acc_ref, *,
              leaky, bm):
    k = pl.program_id(2)

    @pl.when(k == 0)
    def _():
        acc_ref[...] = jnp.zeros_like(acc_ref)

    acc_ref[...] += jnp.dot(a_ref[...], t_ref[...],
                            preferred_element_type=jnp.float32)

    @pl.when(k == pl.num_programs(2) - 1)
    def _():
        i = pl.program_id(0)
        acc = acc_ref[...] + td_ref[...]
        acc = acc * dis_ref[pl.ds(i * bm, bm)][:, None]
        acc = acc + bias_ref[...][None, :]
        if leaky:
            acc = jnp.where(acc > 0, acc, 0.01 * acc)
        out_ref[...] = acc


def _agg(a0, ts, dis, bias, *, leaky, bm=512, bk=512):
    n, d = ts.shape
    grid = (n // bm, 1, n // bk)
    body = functools.partial(_agg_body, leaky=leaky, bm=bm)
    return pl.pallas_call(
        body,
        grid=grid,
        in_specs=[
            pl.BlockSpec((bm, bk), lambda i, j, k: (i, k)),
            pl.BlockSpec((bk, d), lambda i, j, k: (k, j)),
            pl.BlockSpec((bm, d), lambda i, j, k: (i, j)),
            pl.BlockSpec((n,), lambda i, j, k: (0,)),
            pl.BlockSpec((d,), lambda i, j, k: (0,)),
        ],
        out_specs=pl.BlockSpec((bm, d), lambda i, j, k: (i, j)),
        out_shape=jax.ShapeDtypeStruct((n, d), jnp.float32),
        scratch_shapes=[pltpu.VMEM((bm, d), jnp.float32)],
        compiler_params=pltpu.CompilerParams(
            dimension_semantics=("parallel", "parallel", "arbitrary")),
    )(a0, ts, ts, dis, bias)


def _rowscale_body(t_ref, dis_ref, out_ref, *, bm):
    i = pl.program_id(0)
    out_ref[...] = t_ref[...] * dis_ref[pl.ds(i * bm, bm)][:, None]


def _rowscale(t, dis, *, bm=512):
    n, d = t.shape
    return pl.pallas_call(
        functools.partial(_rowscale_body, bm=bm),
        grid=(n // bm,),
        in_specs=[pl.BlockSpec((bm, d), lambda i: (i, 0)),
                  pl.BlockSpec((n,), lambda i: (0,))],
        out_specs=pl.BlockSpec((bm, d), lambda i: (i, 0)),
        out_shape=jax.ShapeDtypeStruct((n, d), jnp.float32),
    )(t, dis)


def _dis_body(a_ref, out_ref):
    out_ref[...] = jax.lax.rsqrt(jnp.sum(a_ref[...], axis=1) + 1.0)


def _dis(a0, *, bm=512):
    n = a0.shape[0]
    return pl.pallas_call(
        _dis_body,
        grid=(n // bm,),
        in_specs=[pl.BlockSpec((bm, n), lambda i: (i, 0))],
        out_specs=pl.BlockSpec((bm,), lambda i: (i,)),
        out_shape=jax.ShapeDtypeStruct((n,), jnp.float32),
    )(a0)


def _conv_pass(t1, a0, params):
    (_, b1, W2, b2, W3, b3, W4, b4, W5, b5, W6, b6) = params
    dis = _dis(a0)
    f = _agg(a0, _rowscale(t1, dis), dis, b1, leaky=False)
    for (W, b) in ((W2, b2), (W3, b3), (W4, b4), (W5, b5)):
        ts = _mm(f, W, scale=dis)
        f = _agg(a0, ts, dis, b, leaky=True)
    g = _mm(a0, f)
    return _mm(g, W6, bias=b6, leaky=True)


_ROWS = 256
_PASSES = N // (_ROWS * 2)
_EPT = E // 16
_WPT = _ROWS * N // 16


def _build_a0(edge_index):
    mesh = plsc.VectorSubcoreMesh(core_axis_name="c", subcore_axis_name="s")

    @functools.partial(
        pl.kernel,
        out_type=jax.ShapeDtypeStruct((N * N,), jnp.float32),
        mesh=mesh,
        scratch_types=[
            pltpu.VMEM_SHARED((_ROWS * N,), jnp.float32),
            pltpu.VMEM((_EPT,), jnp.int32),
            pltpu.VMEM((_EPT,), jnp.int32),
            pltpu.VMEM((_EPT // 128, 128), jnp.int32),
            pltpu.VMEM((_EPT // 128, 128), jnp.float32),
            pltpu.VMEM((8192,), jnp.float32),
        ],
    )
    def k(ei_hbm, a0_hbm, acc, srcv, dstv, idx2d, val2d, zerov):
        c = lax.axis_index("c")
        s = lax.axis_index("s")
        base_e = s * _EPT
        pltpu.sync_copy(ei_hbm.at[pl.ds(base_e, _EPT)], srcv)
        pltpu.sync_copy(ei_hbm.at[pl.ds(E + base_e, _EPT)], dstv)

        def zinit(i, carry):
            zerov[pl.ds(i * 16, 16)] = jnp.zeros((16,), jnp.float32)
            return carry
        lax.fori_loop(0, 8192 // 16, zinit, 0)

        for p in range(_PASSES):
            rb = p * (2 * _ROWS) + c * _ROWS

            def zslice(i, carry):
                pltpu.sync_copy(
                    zerov, acc.at[pl.ds(s * _WPT + i * 8192, 8192)])
                return carry
            lax.fori_loop(0, _WPT // 8192, zslice, 0)
            plsc.subcore_barrier()

            def chunk(j, carry):
                for i in range(8):
                    off = j * 128 + i * 16
                    d = dstv[pl.ds(off, 16)]
                    sv = srcv[pl.ds(off, 16)]
                    rel = d - rb
                    mask = (rel >= 0) & (rel < _ROWS)
                    flat = rel * N + sv
                    spread = off + lax.iota(jnp.int32, 16)
                    idx2d[j, pl.ds(i * 16, 16)] = jnp.where(mask, flat, spread)
                    val2d[j, pl.ds(i * 16, 16)] = jnp.where(
                        mask, jnp.full((16,), 1.0, jnp.float32),
                        jnp.zeros((16,), jnp.float32))
                pltpu.sync_copy(val2d.at[j], acc.at[idx2d.at[j]], add=True)
                return carry
            lax.fori_loop(0, _EPT // 128, chunk, 0)
            plsc.subcore_barrier()

            pltpu.sync_copy(
                acc.at[pl.ds(s * _WPT, _WPT)],
                a0_hbm.at[pl.ds((rb + s * 16) * N, _WPT)])
            plsc.subcore_barrier()

    return k(edge_index.reshape(-1)).reshape(N, N)


def kernel(edge_index_1, edge_index_2, feature, W1, b1, W2, b2, W3, b3,
           W4, b4, W5, b5, W6, b6):
    params = (W1, b1, W2, b2, W3, b3, W4, b4, W5, b5, W6, b6)
    a0_1 = _build_a0(edge_index_1)
    a0_2 = _build_a0(edge_index_2)
    t1 = _mm(feature, W1)
    fa = _conv_pass(t1, a0_1, params)
    fb = _conv_pass(t1, a0_2, params)
    pred = _mm(fa, fb, trans_lhs=True)
    return (fa, fb, pred)


# speedup vs baseline: 14.4102x; 1.0018x over previous
import functools

import jax
import jax.numpy as jnp
from jax import lax
from jax.experimental import pallas as pl
from jax.experimental.pallas import tpu as pltpu
from jax.experimental.pallas import tpu_sc as plsc

N = 4096
D = 512
E = 131072


def _mm_body(a_ref, b_ref, scale_ref, bias_ref, out_ref, acc_ref, *,
             trans_lhs, scale_rows, bias, leaky, bm, bf16):
    k = pl.program_id(2)

    @pl.when(k == 0)
    def _():
        acc_ref[...] = jnp.zeros_like(acc_ref)

    a, b = a_ref[...], b_ref[...]
    if bf16:
        a, b = a.astype(jnp.bfloat16), b.astype(jnp.bfloat16)
    if trans_lhs:
        acc_ref[...] += jax.lax.dot_general(
            a, b, (((0,), (0,)), ((), ())),
            preferred_element_type=jnp.float32)
    else:
        acc_ref[...] += jnp.dot(a, b, preferred_element_type=jnp.float32)

    @pl.when(k == pl.num_programs(2) - 1)
    def _():
        acc = acc_ref[...]
        if bias:
            j = pl.program_id(1)
            bn = out_ref.shape[1]
            acc = acc + bias_ref[pl.ds(j * bn, bn)][None, :]
        if scale_rows:
            i = pl.program_id(0)
            acc = acc * scale_ref[pl.ds(i * bm, bm)][:, None]
        if leaky:
            acc = jnp.where(acc > 0, acc, 0.01 * acc)
        out_ref[...] = acc


def _mm(a, b, *, scale=None, bias=None, leaky=False, trans_lhs=False,
        bf16=False, bm=512, bn=512, bk=512):
    if trans_lhs:
        ka, m = a.shape
    else:
        m, ka = a.shape
    kb, n = b.shape
    assert ka == kb
    grid = (m // bm, n // bn, ka // bk)
    in_specs = [
        pl.BlockSpec((bk, bm) if trans_lhs else (bm, bk),
                     (lambda i, j, k: (k, i)) if trans_lhs
                     else (lambda i, j, k: (i, k))),
        pl.BlockSpec((bk, bn), lambda i, j, k: (k, j)),
        pl.BlockSpec((m,), lambda i, j, k: (0,)),
        pl.BlockSpec((n,), lambda i, j, k: (0,)),
    ]
    scale_arr = scale if scale is not None else jnp.zeros((m,), jnp.float32)
    bias_arr = bias if bias is not None else jnp.zeros((n,), jnp.float32)
    body = functools.partial(_mm_body, trans_lhs=trans_lhs,
                             scale_rows=scale is not None,
                             bias=bias is not None, leaky=leaky, bm=bm,
                             bf16=bf16)
    return pl.pallas_call(
        body,
        grid=grid,
        in_specs=in_specs,
        out_specs=pl.BlockSpec((bm, bn), lambda i, j, k: (i, j)),
        out_shape=jax.ShapeDtypeStruct((m, n), jnp.float32),
        scratch_shapes=[pltpu.VMEM((bm, bn), jnp.float32)],
        compiler_params=pltpu.CompilerParams(
            dimension_semantics=("parallel", "parallel", "arbitrary")),
    )(a, b, scale_arr, bias_arr)


def _agg_body(a_ref, t_ref, td_ref, dis_ref, bias_ref, out_ref, acc_ref, *,
              leaky, bm, bf16):
    k = pl.program_id(2)

    @pl.when(k == 0)
    def _():
        acc_ref[...] = jnp.zeros_like(acc_ref)

    a, t = a_ref[...], t_ref[...]
    if bf16:
        a, t = a.astype(jnp.bfloat16), t.astype(jnp.bfloat16)
    acc_ref[...] += jnp.dot(a, t, preferred_element_type=jnp.float32)

    @pl.when(k == pl.num_programs(2) - 1)
    def _():
        i = pl.program_id(0)
        acc = acc_ref[...] + td_ref[...]
        acc = acc * dis_ref[pl.ds(i * bm, bm)][:, None]
        acc = acc + bias_ref[...][None, :]
        if leaky:
            acc = jnp.where(acc > 0, acc, 0.01 * acc)
        out_ref[...] = acc


def _agg(a0, ts, dis, bias, *, leaky, bf16=False, bm=512, bk=512):
    n, d = ts.shape
    grid = (n // bm, 1, n // bk)
    body = functools.partial(_agg_body, leaky=leaky, bm=bm, bf16=bf16)
    return pl.pallas_call(
        body,
        grid=grid,
        in_specs=[
            pl.BlockSpec((bm, bk), lambda i, j, k: (i, k)),
            pl.BlockSpec((bk, d), lambda i, j, k: (k, j)),
            pl.BlockSpec((bm, d), lambda i, j, k: (i, j)),
            pl.BlockSpec((n,), lambda i, j, k: (0,)),
            pl.BlockSpec((d,), lambda i, j, k: (0,)),
        ],
        out_specs=pl.BlockSpec((bm, d), lambda i, j, k: (i, j)),
        out_shape=jax.ShapeDtypeStruct((n, d), jnp.float32),
        scratch_shapes=[pltpu.VMEM((bm, d), jnp.float32)],
        compiler_params=pltpu.CompilerParams(
            dimension_semantics=("parallel", "parallel", "arbitrary")),
    )(a0, ts, ts, dis, bias)


def _rowscale_body(t_ref, dis_ref, out_ref, *, bm):
    i = pl.program_id(0)
    out_ref[...] = t_ref[...] * dis_ref[pl.ds(i * bm, bm)][:, None]


def _rowscale(t, dis, *, bm=512):
    n, d = t.shape
    return pl.pallas_call(
        functools.partial(_rowscale_body, bm=bm),
        grid=(n // bm,),
        in_specs=[pl.BlockSpec((bm, d), lambda i: (i, 0)),
                  pl.BlockSpec((n,), lambda i: (0,))],
        out_specs=pl.BlockSpec((bm, d), lambda i: (i, 0)),
        out_shape=jax.ShapeDtypeStruct((n, d), jnp.float32),
    )(t, dis)


def _dis_body(a_ref, out_ref):
    out_ref[...] = jax.lax.rsqrt(jnp.sum(a_ref[...], axis=1) + 1.0)


def _dis(a0, *, bm=512):
    n = a0.shape[0]
    return pl.pallas_call(
        _dis_body,
        grid=(n // bm,),
        in_specs=[pl.BlockSpec((bm, n), lambda i: (i, 0))],
        out_specs=pl.BlockSpec((bm,), lambda i: (i,)),
        out_shape=jax.ShapeDtypeStruct((n,), jnp.float32),
    )(a0)


def _conv_pass(t1, a0, params):
    (_, b1, W2, b2, W3, b3, W4, b4, W5, b5, W6, b6) = params
    dis = _dis(a0)
    f = _agg(a0, _rowscale(t1, dis), dis, b1, leaky=False, bf16=True)
    for (W, b) in ((W2, b2), (W3, b3), (W4, b4), (W5, b5)):
        ts = _mm(f, W, scale=dis, bf16=True)
        f = _agg(a0, ts, dis, b, leaky=True, bf16=True)
    g = _mm(a0, f, bf16=True)
    return _mm(g, W6, bias=b6, leaky=True, bf16=True)


_ROWS = 256
_PASSES = N // (_ROWS * 2)
_EPT = E // 16
_WPT = _ROWS * N // 16


def _build_a0(edge_index):
    mesh = plsc.VectorSubcoreMesh(core_axis_name="c", subcore_axis_name="s")

    @functools.partial(
        pl.kernel,
        out_type=jax.ShapeDtypeStruct((N * N,), jnp.float32),
        mesh=mesh,
        scratch_types=[
            pltpu.VMEM_SHARED((_ROWS * N,), jnp.float32),
            pltpu.VMEM((_EPT,), jnp.int32),
            pltpu.VMEM((_EPT,), jnp.int32),
            pltpu.VMEM((_EPT // 128, 128), jnp.int32),
            pltpu.VMEM((_EPT // 128, 128), jnp.float32),
            pltpu.VMEM((8192,), jnp.float32),
        ],
    )
    def k(ei_hbm, a0_hbm, acc, srcv, dstv, idx2d, val2d, zerov):
        c = lax.axis_index("c")
        s = lax.axis_index("s")
        base_e = s * _EPT
        pltpu.sync_copy(ei_hbm.at[pl.ds(base_e, _EPT)], srcv)
        pltpu.sync_copy(ei_hbm.at[pl.ds(E + base_e, _EPT)], dstv)

        def zinit(i, carry):
            zerov[pl.ds(i * 16, 16)] = jnp.zeros((16,), jnp.float32)
            return carry
        lax.fori_loop(0, 8192 // 16, zinit, 0)

        for p in range(_PASSES):
            rb = p * (2 * _ROWS) + c * _ROWS

            def zslice(i, carry):
                pltpu.sync_copy(
                    zerov, acc.at[pl.ds(s * _WPT + i * 8192, 8192)])
                return carry
            lax.fori_loop(0, _WPT // 8192, zslice, 0)
            plsc.subcore_barrier()

            def chunk(j, carry):
                for i in range(8):
                    off = j * 128 + i * 16
                    d = dstv[pl.ds(off, 16)]
                    sv = srcv[pl.ds(off, 16)]
                    rel = d - rb
                    mask = (rel >= 0) & (rel < _ROWS)
                    flat = rel * N + sv
                    spread = off + lax.iota(jnp.int32, 16)
                    idx2d[j, pl.ds(i * 16, 16)] = jnp.where(mask, flat, spread)
                    val2d[j, pl.ds(i * 16, 16)] = jnp.where(
                        mask, jnp.full((16,), 1.0, jnp.float32),
                        jnp.zeros((16,), jnp.float32))
                pltpu.sync_copy(val2d.at[j], acc.at[idx2d.at[j]], add=True)
                return carry
            lax.fori_loop(0, _EPT // 128, chunk, 0)
            plsc.subcore_barrier()

            pltpu.sync_copy(
                acc.at[pl.ds(s * _WPT, _WPT)],
                a0_hbm.at[pl.ds((rb + s * 16) * N, _WPT)])
            plsc.subcore_barrier()

    return k(edge_index.reshape(-1)).reshape(N, N)


def kernel(edge_index_1, edge_index_2, feature, W1, b1, W2, b2, W3, b3,
           W4, b4, W5, b5, W6, b6):
    params = (W1, b1, W2, b2, W3, b3, W4, b4, W5, b5, W6, b6)
    a0_1 = _build_a0(edge_index_1)
    a0_2 = _build_a0(edge_index_2)
    t1 = _mm(feature, W1, bf16=True)
    fa = _conv_pass(t1, a0_1, params)
    fb = _conv_pass(t1, a0_2, params)
    pred = _mm(fa, fb, trans_lhs=True, bf16=True)
    return (fa, fb, pred)


# speedup vs baseline: 25.0604x; 1.7391x over previous
import functools

import jax
import jax.numpy as jnp
from jax import lax
from jax.experimental import pallas as pl
from jax.experimental.pallas import tpu as pltpu
from jax.experimental.pallas import tpu_sc as plsc

N = 4096
D = 512
E = 131072


def _mm_body(a_ref, b_ref, scale_ref, bias_ref, out_ref, acc_ref, *,
             trans_lhs, scale_rows, bias, leaky, bm, bf16):
    k = pl.program_id(2)

    @pl.when(k == 0)
    def _():
        acc_ref[...] = jnp.zeros_like(acc_ref)

    a, b = a_ref[...], b_ref[...]
    if bf16:
        a, b = a.astype(jnp.bfloat16), b.astype(jnp.bfloat16)
    if trans_lhs:
        acc_ref[...] += jax.lax.dot_general(
            a, b, (((0,), (0,)), ((), ())),
            preferred_element_type=jnp.float32)
    else:
        acc_ref[...] += jnp.dot(a, b, preferred_element_type=jnp.float32)

    @pl.when(k == pl.num_programs(2) - 1)
    def _():
        acc = acc_ref[...]
        if bias:
            j = pl.program_id(1)
            bn = out_ref.shape[1]
            acc = acc + bias_ref[pl.ds(j * bn, bn)][None, :]
        if scale_rows:
            i = pl.program_id(0)
            acc = acc * scale_ref[pl.ds(i * bm, bm)][:, None]
        if leaky:
            acc = jnp.where(acc > 0, acc, 0.01 * acc)
        out_ref[...] = acc


def _mm(a, b, *, scale=None, bias=None, leaky=False, trans_lhs=False,
        bf16=False, bm=512, bn=512, bk=512):
    if trans_lhs:
        ka, m = a.shape
    else:
        m, ka = a.shape
    kb, n = b.shape
    assert ka == kb
    grid = (m // bm, n // bn, ka // bk)
    in_specs = [
        pl.BlockSpec((bk, bm) if trans_lhs else (bm, bk),
                     (lambda i, j, k: (k, i)) if trans_lhs
                     else (lambda i, j, k: (i, k))),
        pl.BlockSpec((bk, bn), lambda i, j, k: (k, j)),
        pl.BlockSpec((m,), lambda i, j, k: (0,)),
        pl.BlockSpec((n,), lambda i, j, k: (0,)),
    ]
    scale_arr = scale if scale is not None else jnp.zeros((m,), jnp.float32)
    bias_arr = bias if bias is not None else jnp.zeros((n,), jnp.float32)
    body = functools.partial(_mm_body, trans_lhs=trans_lhs,
                             scale_rows=scale is not None,
                             bias=bias is not None, leaky=leaky, bm=bm,
                             bf16=bf16)
    return pl.pallas_call(
        body,
        grid=grid,
        in_specs=in_specs,
        out_specs=pl.BlockSpec((bm, bn), lambda i, j, k: (i, j)),
        out_shape=jax.ShapeDtypeStruct((m, n), jnp.float32),
        scratch_shapes=[pltpu.VMEM((bm, bn), jnp.float32)],
        compiler_params=pltpu.CompilerParams(
            dimension_semantics=("parallel", "parallel", "arbitrary")),
    )(a, b, scale_arr, bias_arr)


def _agg_body(*refs, norm, leaky, bf16):
    if norm:
        a_ref, t_ref, td_ref, dis_ref, bias_ref, out_ref, acc_ref = refs
    else:
        a_ref, t_ref, out_ref, acc_ref = refs
    k = pl.program_id(0)

    @pl.when(k == 0)
    def _():
        acc_ref[...] = jnp.zeros_like(acc_ref)

    a, t = a_ref[...], t_ref[...]
    if bf16:
        a, t = a.astype(jnp.bfloat16), t.astype(jnp.bfloat16)
    acc_ref[...] += jnp.dot(a, t, preferred_element_type=jnp.float32)

    @pl.when(k == pl.num_programs(0) - 1)
    def _():
        acc = acc_ref[...]
        if norm:
            acc = (acc + td_ref[...]) * dis_ref[...][:, None]
            acc = acc + bias_ref[...][None, :]
        if leaky:
            acc = jnp.where(acc > 0, acc, 0.01 * acc)
        out_ref[...] = acc


def _agg(a0, ts, dis=None, bias=None, *, leaky=False, bf16=False, bk=512):
    n, d = ts.shape
    norm = dis is not None
    body = functools.partial(_agg_body, norm=norm, leaky=leaky, bf16=bf16)
    in_specs = [
        pl.BlockSpec((n, bk), lambda k: (0, k)),
        pl.BlockSpec((bk, d), lambda k: (k, 0)),
    ]
    args = [a0, ts]
    if norm:
        in_specs += [
            pl.BlockSpec((n, d), lambda k: (0, 0)),
            pl.BlockSpec((n,), lambda k: (0,)),
            pl.BlockSpec((d,), lambda k: (0,)),
        ]
        args += [ts, dis, bias]
    return pl.pallas_call(
        body,
        grid=(n // bk,),
        in_specs=in_specs,
        out_specs=pl.BlockSpec((n, d), lambda k: (0, 0)),
        out_shape=jax.ShapeDtypeStruct((n, d), jnp.float32),
        scratch_shapes=[pltpu.VMEM((n, d), jnp.float32)],
        compiler_params=pltpu.CompilerParams(
            dimension_semantics=("arbitrary",)),
    )(*args)


def _rowscale_body(t_ref, dis_ref, out_ref, *, bm):
    i = pl.program_id(0)
    out_ref[...] = t_ref[...] * dis_ref[pl.ds(i * bm, bm)][:, None]


def _rowscale(t, dis, *, bm=512):
    n, d = t.shape
    return pl.pallas_call(
        functools.partial(_rowscale_body, bm=bm),
        grid=(n // bm,),
        in_specs=[pl.BlockSpec((bm, d), lambda i: (i, 0)),
                  pl.BlockSpec((n,), lambda i: (0,))],
        out_specs=pl.BlockSpec((bm, d), lambda i: (i, 0)),
        out_shape=jax.ShapeDtypeStruct((n, d), jnp.float32),
    )(t, dis)


def _dis_body(a_ref, out_ref):
    out_ref[...] = jax.lax.rsqrt(jnp.sum(a_ref[...], axis=1) + 1.0)


def _dis(a0, *, bm=512):
    n = a0.shape[0]
    return pl.pallas_call(
        _dis_body,
        grid=(n // bm,),
        in_specs=[pl.BlockSpec((bm, n), lambda i: (i, 0))],
        out_specs=pl.BlockSpec((bm,), lambda i: (i,)),
        out_shape=jax.ShapeDtypeStruct((n,), jnp.float32),
    )(a0)


def _conv_pass(t1, a0, params):
    (_, b1, W2, b2, W3, b3, W4, b4, W5, b5, W6, b6) = params
    dis = _dis(a0)
    f = _agg(a0, _rowscale(t1, dis), dis, b1, leaky=False, bf16=True)
    for (W, b) in ((W2, b2), (W3, b3), (W4, b4), (W5, b5)):
        ts = _mm(f, W, scale=dis, bf16=True)
        f = _agg(a0, ts, dis, b, leaky=True, bf16=True)
    g = _agg(a0, f, bf16=True)
    return _mm(g, W6, bias=b6, leaky=True, bf16=True, bm=1024, bn=1024)


_ROWS = 256
_PASSES = N // (_ROWS * 2)
_EPT = E // 16
_WPT = _ROWS * N // 16


def _build_a0(edge_index):
    mesh = plsc.VectorSubcoreMesh(core_axis_name="c", subcore_axis_name="s")

    @functools.partial(
        pl.kernel,
        out_type=jax.ShapeDtypeStruct((N * N,), jnp.float32),
        mesh=mesh,
        scratch_types=[
            pltpu.VMEM_SHARED((_ROWS * N,), jnp.float32),
            pltpu.VMEM((_EPT,), jnp.int32),
            pltpu.VMEM((_EPT,), jnp.int32),
            pltpu.VMEM((_EPT // 128, 128), jnp.int32),
            pltpu.VMEM((_EPT // 128, 128), jnp.float32),
            pltpu.VMEM((8192,), jnp.float32),
        ],
    )
    def k(ei_hbm, a0_hbm, acc, srcv, dstv, idx2d, val2d, zerov):
        c = lax.axis_index("c")
        s = lax.axis_index("s")
        base_e = s * _EPT
        pltpu.sync_copy(ei_hbm.at[pl.ds(base_e, _EPT)], srcv)
        pltpu.sync_copy(ei_hbm.at[pl.ds(E + base_e, _EPT)], dstv)

        def zinit(i, carry):
            zerov[pl.ds(i * 16, 16)] = jnp.zeros((16,), jnp.float32)
            return carry
        lax.fori_loop(0, 8192 // 16, zinit, 0)

        for p in range(_PASSES):
            rb = p * (2 * _ROWS) + c * _ROWS

            def zslice(i, carry):
                pltpu.sync_copy(
                    zerov, acc.at[pl.ds(s * _WPT + i * 8192, 8192)])
                return carry
            lax.fori_loop(0, _WPT // 8192, zslice, 0)
            plsc.subcore_barrier()

            def chunk(j, carry):
                for i in range(8):
                    off = j * 128 + i * 16
                    d = dstv[pl.ds(off, 16)]
                    sv = srcv[pl.ds(off, 16)]
                    rel = d - rb
                    mask = (rel >= 0) & (rel < _ROWS)
                    flat = rel * N + sv
                    spread = off + lax.iota(jnp.int32, 16)
                    idx2d[j, pl.ds(i * 16, 16)] = jnp.where(mask, flat, spread)
                    val2d[j, pl.ds(i * 16, 16)] = jnp.where(
                        mask, jnp.full((16,), 1.0, jnp.float32),
                        jnp.zeros((16,), jnp.float32))
                pltpu.sync_copy(val2d.at[j], acc.at[idx2d.at[j]], add=True)
                return carry
            lax.fori_loop(0, _EPT // 128, chunk, 0)
            plsc.subcore_barrier()

            pltpu.sync_copy(
                acc.at[pl.ds(s * _WPT, _WPT)],
                a0_hbm.at[pl.ds((rb + s * 16) * N, _WPT)])
            plsc.subcore_barrier()

    return k(edge_index.reshape(-1)).reshape(N, N)


def kernel(edge_index_1, edge_index_2, feature, W1, b1, W2, b2, W3, b3,
           W4, b4, W5, b5, W6, b6):
    params = (W1, b1, W2, b2, W3, b3, W4, b4, W5, b5, W6, b6)
    a0_1 = _build_a0(edge_index_1)
    a0_2 = _build_a0(edge_index_2)
    t1 = _mm(feature, W1, bf16=True)
    fa = _conv_pass(t1, a0_1, params)
    fb = _conv_pass(t1, a0_2, params)
    pred = _mm(fa, fb, trans_lhs=True, bf16=True, bm=1024, bn=1024)
    return (fa, fb, pred)
